# baseline (speedup 1.0000x reference)
import functools

import jax
import jax.numpy as jnp
from jax import lax
from jax.experimental import pallas as pl
from jax.experimental.pallas import tpu as pltpu
from jax.experimental.pallas import tpu_sc as plsc

_VAR0, _VAR1 = 0.1, 0.2
_CONF_THR = 0.5
_NMS_THR = 0.4
_TOP_K = 750
_NP = 20000
_PP = 20480
_R, _L = 160, 128
_SIZE = 640.0
_BIG = 2 ** 30
_KB = 768
_KD = 128


def _decode_nms_body(in_ref, comp_ref, ki_ref, miota_ref,
                     kx1_ref, ky1_ref, kx2_ref, ky2_ref, kta_ref):
    lx, ly, lw, lh = in_ref[0], in_ref[1], in_ref[2], in_ref[3]
    pcx, pcy, pw, ph = in_ref[4], in_ref[5], in_ref[6], in_ref[7]
    conf1 = in_ref[8]

    cx = pcx + lx * _VAR0 * pw
    cy = pcy + ly * _VAR0 * ph
    w = pw * jnp.exp(lw * _VAR1)
    h = ph * jnp.exp(lh * _VAR1)
    x1 = (cx - w / 2.0) * _SIZE
    y1 = (cy - h / 2.0) * _SIZE
    x2 = (cx + w / 2.0) * _SIZE
    y2 = (cy + h / 2.0) * _SIZE
    comp_ref[0] = x1
    comp_ref[1] = y1
    comp_ref[2] = x2
    comp_ref[3] = y2
    for k in range(5):
        comp_ref[4 + 2 * k] = (pcx + pw * in_ref[9 + 2 * k] * _VAR0) * _SIZE
        comp_ref[5 + 2 * k] = (pcy + ph * in_ref[9 + 2 * k + 1] * _VAR0) * _SIZE

    c_thr = _NMS_THR / (1.0 + _NMS_THR)
    c_eps = 1e-9 * c_thr

    iota2 = (lax.broadcasted_iota(jnp.int32, (_R, _L), 0) * _L
             + lax.broadcasted_iota(jnp.int32, (_R, _L), 1))
    lane_iota = lax.broadcasted_iota(jnp.int32, (1, _L), 1)
    kio = (lax.broadcasted_iota(jnp.int32, (8, 128), 0) * 128
           + lax.broadcasted_iota(jnp.int32, (8, 128), 1))

    miota_ref[...] = jnp.where(conf1 > _CONF_THR, iota2, _BIG)

    def ext_sel(first):
        row = first // _L
        lane = first - row * _L

        def ext(c):
            v = comp_ref[c, pl.ds(row, 1), :]
            return jnp.sum(jnp.where(lane_iota == lane, v, 0.0))

        return ext(0), ext(1), ext(2), ext(3)

    def chunk_body(carry):
        fc, s, nk, keep = carry
        rb = fc * 8
        x1c = comp_ref[0, pl.ds(rb, 8), :]
        y1c = comp_ref[1, pl.ds(rb, 8), :]
        x2c = comp_ref[2, pl.ds(rb, 8), :]
        y2c = comp_ref[3, pl.ds(rb, 8), :]
        tac = (jnp.maximum(x2c - x1c, 0.0) * jnp.maximum(y2c - y1c, 0.0)
               * c_thr + c_eps)

        def entry_test(k, cm):
            xi1, yi1 = kx1_ref[k], ky1_ref[k]
            xi2, yi2 = kx2_ref[k], ky2_ref[k]
            inter = (jnp.maximum(jnp.minimum(xi2, x2c) - jnp.maximum(xi1, x1c), 0.0)
                     * jnp.maximum(jnp.minimum(yi2, y2c) - jnp.maximum(yi1, y1c), 0.0))
            return jnp.where(inter > tac + kta_ref[k], _BIG, cm)

        cm0 = lax.fori_loop(0, nk, entry_test, miota_ref[pl.ds(rb, 8), :])

        def sel_cond(c):
            s, nk, first, cm, keep = c
            return jnp.logical_and(s < _TOP_K, first < _PP)

        def sel_body(c):
            s, nk, first, cm, keep = c
            xi1, yi1, xi2, yi2 = ext_sel(first)
            ai = jnp.maximum(xi2 - xi1, 0.0) * jnp.maximum(yi2 - yi1, 0.0)
            kx1_ref[nk] = xi1
            ky1_ref[nk] = yi1
            kx2_ref[nk] = xi2
            ky2_ref[nk] = yi2
            kta_ref[nk] = ai * c_thr
            inter = (jnp.maximum(jnp.minimum(xi2, x2c) - jnp.maximum(xi1, x1c), 0.0)
                     * jnp.maximum(jnp.minimum(yi2, y2c) - jnp.maximum(yi1, y1c), 0.0))
            kill = (inter > tac + ai * c_thr) | (cm == first)
            cm = jnp.where(kill, _BIG, cm)
            keep = jnp.where(kio == s, first, keep)
            return s + 1, nk + 1, jnp.min(cm), cm, keep

        s, nk, _, _, keep = lax.while_loop(
            sel_cond, sel_body, (s, nk, jnp.min(cm0), cm0, keep))
        return fc + 1, s, nk, keep

    def chunk_cond(carry):
        fc, s, _, _ = carry
        return jnp.logical_and(fc < _R // 8, s < _TOP_K)

    _, _, _, keep = lax.while_loop(
        chunk_cond, chunk_body,
        (0, 0, 0, jnp.full((8, 128), _NP, jnp.int32)))
    ki_ref[...] = keep


def _sc_gather(table, idx):
    info = plsc.get_sparse_core_info()
    nw = info.num_cores * info.num_subcores
    bpw = _KB // nw
    mesh = plsc.VectorSubcoreMesh(core_axis_name="c", subcore_axis_name="s")

    @functools.partial(
        pl.kernel, mesh=mesh,
        out_type=jax.ShapeDtypeStruct((_KB, _KD), jnp.float32),
        scratch_types=[
            pltpu.VMEM((bpw,), jnp.int32),
            pltpu.VMEM((bpw, _KD), jnp.float32),
            pltpu.SemaphoreType.DMA,
        ],
    )
    def k(table_hbm, idx_hbm, out_hbm, idx_v, rows_v, sem):
        wid = lax.axis_index("s") * info.num_cores + lax.axis_index("c")
        base = wid * bpw
        pltpu.sync_copy(idx_hbm.at[pl.ds(base, bpw)], idx_v)
        pltpu.async_copy(table_hbm.at[idx_v], rows_v, sem).wait()
        pltpu.sync_copy(rows_v, out_hbm.at[pl.ds(base, bpw)])

    return k(table, idx)


def _comps(a):
    return a.T.reshape(a.shape[1], _R, _L)


def kernel(loc, conf, landms, prior_box):
    pad = _PP - _NP
    locp = jnp.pad(loc[0], ((0, pad), (0, 0)))
    confp = jnp.pad(conf[0], ((0, pad), (0, 0)))
    lmp = jnp.pad(landms[0], ((0, pad), (0, 0)))
    prp = jnp.pad(prior_box, ((0, pad), (0, 0)))
    confT = _comps(confp)
    inp = jnp.concatenate(
        [_comps(locp), _comps(prp), confT[1:2], _comps(lmp)], axis=0)

    comp, ki = pl.pallas_call(
        _decode_nms_body,
        out_shape=[
            jax.ShapeDtypeStruct((14, _R, _L), jnp.float32),
            jax.ShapeDtypeStruct((8, 128), jnp.int32),
        ],
        scratch_shapes=[pltpu.VMEM((_R, _L), jnp.int32)]
        + [pltpu.SMEM((_KB,), jnp.float32)] * 5,
    )(inp)

    table = jnp.concatenate(
        [confT, comp, jnp.zeros((_KD - 16, _R, _L), jnp.float32)],
        axis=0).reshape(_KD, _PP).T
    idx = ki.reshape(-1)[:_KB]
    rows = _sc_gather(table, idx)
    conf_out = rows[:_TOP_K, 0:2]
    loc_out = rows[:_TOP_K, 2:6]
    lm_out = rows[:_TOP_K, 6:16]
    return conf_out, lm_out, loc_out


# speedup vs baseline: 1.0914x; 1.0914x over previous
import functools

import jax
import jax.numpy as jnp
from jax import lax
from jax.experimental import pallas as pl
from jax.experimental.pallas import tpu as pltpu
from jax.experimental.pallas import tpu_sc as plsc

_VAR0, _VAR1 = 0.1, 0.2
_CONF_THR = 0.5
_NMS_THR = 0.4
_TOP_K = 750
_NP = 20000
_PP = 20480
_R, _L = 160, 128
_SIZE = 640.0
_BIG = 2 ** 30
_KB = 768
_KD = 128


def _decode_nms_body(in_ref, comp_ref, ki_ref, miota_ref,
                     kx1_ref, ky1_ref, kx2_ref, ky2_ref, kta_ref):
    lx, ly, lw, lh = in_ref[0], in_ref[1], in_ref[2], in_ref[3]
    pcx, pcy, pw, ph = in_ref[4], in_ref[5], in_ref[6], in_ref[7]
    conf1 = in_ref[8]

    cx = pcx + lx * _VAR0 * pw
    cy = pcy + ly * _VAR0 * ph
    w = pw * jnp.exp(lw * _VAR1)
    h = ph * jnp.exp(lh * _VAR1)
    x1 = (cx - w / 2.0) * _SIZE
    y1 = (cy - h / 2.0) * _SIZE
    x2 = (cx + w / 2.0) * _SIZE
    y2 = (cy + h / 2.0) * _SIZE
    comp_ref[0] = x1
    comp_ref[1] = y1
    comp_ref[2] = x2
    comp_ref[3] = y2
    for k in range(5):
        comp_ref[4 + 2 * k] = (pcx + pw * in_ref[9 + 2 * k] * _VAR0) * _SIZE
        comp_ref[5 + 2 * k] = (pcy + ph * in_ref[9 + 2 * k + 1] * _VAR0) * _SIZE

    c_thr = _NMS_THR / (1.0 + _NMS_THR)
    c_eps = 1e-9 * c_thr

    iota2 = (lax.broadcasted_iota(jnp.int32, (_R, _L), 0) * _L
             + lax.broadcasted_iota(jnp.int32, (_R, _L), 1))
    lane_iota = lax.broadcasted_iota(jnp.int32, (1, _L), 1)
    kio = (lax.broadcasted_iota(jnp.int32, (8, 128), 0) * 128
           + lax.broadcasted_iota(jnp.int32, (8, 128), 1))

    miota_ref[...] = jnp.where(conf1 > _CONF_THR, iota2, _BIG)

    def ext_sel(first):
        row = first // _L
        lane = first - row * _L

        def ext(c):
            v = comp_ref[c, pl.ds(row, 1), :]
            return jnp.sum(jnp.where(lane_iota == lane, v, 0.0))

        return ext(0), ext(1), ext(2), ext(3)

    def chunk_body(carry):
        fc, s, nk, keep = carry
        rb = fc * 8
        x1c = comp_ref[0, pl.ds(rb, 8), :]
        y1c = comp_ref[1, pl.ds(rb, 8), :]
        x2c = comp_ref[2, pl.ds(rb, 8), :]
        y2c = comp_ref[3, pl.ds(rb, 8), :]
        tac = (jnp.maximum(x2c - x1c, 0.0) * jnp.maximum(y2c - y1c, 0.0)
               * c_thr + c_eps)

        def entry_test(k, cm):
            xi1, yi1 = kx1_ref[k], ky1_ref[k]
            xi2, yi2 = kx2_ref[k], ky2_ref[k]
            inter = (jnp.maximum(jnp.minimum(xi2, x2c) - jnp.maximum(xi1, x1c), 0.0)
                     * jnp.maximum(jnp.minimum(yi2, y2c) - jnp.maximum(yi1, y1c), 0.0))
            return jnp.where(inter > tac + kta_ref[k], _BIG, cm)

        cm0 = lax.fori_loop(0, nk, entry_test, miota_ref[pl.ds(rb, 8), :])

        def sel_cond(c):
            s, nk, first, cm, keep = c
            return jnp.logical_and(s < _TOP_K, first < _PP)

        def sel_body(c):
            s, nk, first, cm, keep = c
            xi1, yi1, xi2, yi2 = ext_sel(first)
            ai = jnp.maximum(xi2 - xi1, 0.0) * jnp.maximum(yi2 - yi1, 0.0)
            kx1_ref[nk] = xi1
            ky1_ref[nk] = yi1
            kx2_ref[nk] = xi2
            ky2_ref[nk] = yi2
            kta_ref[nk] = ai * c_thr
            inter = (jnp.maximum(jnp.minimum(xi2, x2c) - jnp.maximum(xi1, x1c), 0.0)
                     * jnp.maximum(jnp.minimum(yi2, y2c) - jnp.maximum(yi1, y1c), 0.0))
            kill = (inter > tac + ai * c_thr) | (cm == first)
            cm = jnp.where(kill, _BIG, cm)
            keep = jnp.where(kio == s, first, keep)
            return s + 1, nk + 1, jnp.min(cm), cm, keep

        s, nk, _, _, keep = lax.while_loop(
            sel_cond, sel_body, (s, nk, jnp.min(cm0), cm0, keep))
        return fc + 1, s, nk, keep

    def chunk_cond(carry):
        fc, s, _, _ = carry
        return jnp.logical_and(fc < _R // 8, s < _TOP_K)

    _, _, _, keep = lax.while_loop(
        chunk_cond, chunk_body,
        (0, 0, 0, jnp.full((8, 128), _NP, jnp.int32)))
    ki_ref[...] = keep


def _sc_gather(table, idx):
    info = plsc.get_sparse_core_info()
    nw = info.num_cores * info.num_subcores
    bpw = _KB // nw
    mesh = plsc.VectorSubcoreMesh(core_axis_name="c", subcore_axis_name="s")

    @functools.partial(
        pl.kernel, mesh=mesh,
        out_type=jax.ShapeDtypeStruct((_KB, _KD), jnp.float32),
        scratch_types=[
            pltpu.VMEM((bpw,), jnp.int32),
            pltpu.VMEM((bpw, _KD), jnp.float32),
            pltpu.SemaphoreType.DMA,
        ],
    )
    def k(table_hbm, idx_hbm, out_hbm, idx_v, rows_v, sem):
        wid = lax.axis_index("s") * info.num_cores + lax.axis_index("c")
        base = wid * bpw
        pltpu.sync_copy(idx_hbm.at[pl.ds(base, bpw)], idx_v)
        pltpu.async_copy(table_hbm.at[idx_v], rows_v, sem).wait()
        pltpu.sync_copy(rows_v, out_hbm.at[pl.ds(base, bpw)])

    return k(table, idx)


def _comps(a):
    return a.T.reshape(a.shape[1], _R, _L)


def kernel(loc, conf, landms, prior_box):
    pad = _PP - _NP
    locp = jnp.pad(loc[0], ((0, pad), (0, 0)))
    confp = jnp.pad(conf[0], ((0, pad), (0, 0)))
    lmp = jnp.pad(landms[0], ((0, pad), (0, 0)))
    prp = jnp.pad(prior_box, ((0, pad), (0, 0)))
    confT = _comps(confp)
    inp = jnp.concatenate(
        [_comps(locp), _comps(prp), confT[1:2], _comps(lmp)], axis=0)

    comp, ki = pl.pallas_call(
        _decode_nms_body,
        out_shape=[
            jax.ShapeDtypeStruct((14, _R, _L), jnp.float32),
            jax.ShapeDtypeStruct((8, 128), jnp.int32),
        ],
        scratch_shapes=[pltpu.VMEM((_R, _L), jnp.int32)]
        + [pltpu.SMEM((_KB,), jnp.float32)] * 5,
    )(inp)

    table = jnp.concatenate([confT, comp], axis=0).reshape(
        16, _PP).T.reshape(_PP // 8, 128)
    idx = ki.reshape(-1)[:_KB]
    rows = _sc_gather(table, idx >> 3)
    sub = idx & 7
    r8 = rows.reshape(_KB, 8, 16)
    oh = (sub[:, None] == jnp.arange(8)[None, :]).astype(jnp.float32)
    sel = jnp.sum(r8 * oh[:, :, None], axis=1)
    conf_out = sel[:_TOP_K, 0:2]
    loc_out = sel[:_TOP_K, 2:6]
    lm_out = sel[:_TOP_K, 6:16]
    return conf_out, lm_out, loc_out


# speedup vs baseline: 1.1819x; 1.0829x over previous
import functools

import jax
import jax.numpy as jnp
from jax import lax
from jax.experimental import pallas as pl
from jax.experimental.pallas import tpu as pltpu
from jax.experimental.pallas import tpu_sc as plsc

_VAR0, _VAR1 = 0.1, 0.2
_CONF_THR = 0.5
_NMS_THR = 0.4
_TOP_K = 750
_NP = 20000
_PP = 20480
_R, _L = 160, 128
_SIZE = 640.0
_BIG = 2 ** 30
_KB = 768
_KD = 128


def _decode_nms_body(in_ref, comp_ref, ki_ref, miota_ref,
                     kx1_ref, ky1_ref, kx2_ref, ky2_ref, kta_ref):
    lx, ly, lw, lh = in_ref[0], in_ref[1], in_ref[2], in_ref[3]
    pcx, pcy, pw, ph = in_ref[4], in_ref[5], in_ref[6], in_ref[7]
    conf1 = in_ref[8]

    cx = pcx + lx * _VAR0 * pw
    cy = pcy + ly * _VAR0 * ph
    w = pw * jnp.exp(lw * _VAR1)
    h = ph * jnp.exp(lh * _VAR1)
    x1 = (cx - w / 2.0) * _SIZE
    y1 = (cy - h / 2.0) * _SIZE
    x2 = (cx + w / 2.0) * _SIZE
    y2 = (cy + h / 2.0) * _SIZE
    comp_ref[0] = x1
    comp_ref[1] = y1
    comp_ref[2] = x2
    comp_ref[3] = y2
    for k in range(5):
        comp_ref[4 + 2 * k] = (pcx + pw * in_ref[9 + 2 * k] * _VAR0) * _SIZE
        comp_ref[5 + 2 * k] = (pcy + ph * in_ref[9 + 2 * k + 1] * _VAR0) * _SIZE

    c_thr = _NMS_THR / (1.0 + _NMS_THR)
    c_eps = 1e-9 * c_thr

    iota2 = (lax.broadcasted_iota(jnp.int32, (_R, _L), 0) * _L
             + lax.broadcasted_iota(jnp.int32, (_R, _L), 1))
    lane_iota = lax.broadcasted_iota(jnp.int32, (1, _L), 1)
    kio = (lax.broadcasted_iota(jnp.int32, (8, 128), 0) * 128
           + lax.broadcasted_iota(jnp.int32, (8, 128), 1))

    miota_ref[...] = jnp.where(conf1 > _CONF_THR, iota2, _BIG)

    def ext_sel(first):
        row = first // _L
        lane = first - row * _L

        def ext(c):
            v = comp_ref[c, pl.ds(row, 1), :]
            return jnp.sum(jnp.where(lane_iota == lane, v, 0.0))

        return ext(0), ext(1), ext(2), ext(3)

    def chunk_body(carry):
        fc, s, nk, keep = carry
        rb = fc * 8
        x1c = comp_ref[0, pl.ds(rb, 8), :]
        y1c = comp_ref[1, pl.ds(rb, 8), :]
        x2c = comp_ref[2, pl.ds(rb, 8), :]
        y2c = comp_ref[3, pl.ds(rb, 8), :]
        tac = (jnp.maximum(x2c - x1c, 0.0) * jnp.maximum(y2c - y1c, 0.0)
               * c_thr + c_eps)

        def entry_test4(i, cm):
            for j in range(4):
                k = jnp.minimum(4 * i + j, nk - 1)
                xi1, yi1 = kx1_ref[k], ky1_ref[k]
                xi2, yi2 = kx2_ref[k], ky2_ref[k]
                inter = (jnp.maximum(jnp.minimum(xi2, x2c) - jnp.maximum(xi1, x1c), 0.0)
                         * jnp.maximum(jnp.minimum(yi2, y2c) - jnp.maximum(yi1, y1c), 0.0))
                cm = jnp.where(inter > tac + kta_ref[k], _BIG, cm)
            return cm

        cm0 = lax.fori_loop(0, (nk + 3) // 4, entry_test4,
                            miota_ref[pl.ds(rb, 8), :])

        def sel_cond(c):
            s, nk, first, cm, keep = c
            return jnp.logical_and(s < _TOP_K, first < _PP)

        def sel_body(c):
            s, nk, first, cm, keep = c
            xi1, yi1, xi2, yi2 = ext_sel(first)
            ai = jnp.maximum(xi2 - xi1, 0.0) * jnp.maximum(yi2 - yi1, 0.0)
            kx1_ref[nk] = xi1
            ky1_ref[nk] = yi1
            kx2_ref[nk] = xi2
            ky2_ref[nk] = yi2
            kta_ref[nk] = ai * c_thr
            inter = (jnp.maximum(jnp.minimum(xi2, x2c) - jnp.maximum(xi1, x1c), 0.0)
                     * jnp.maximum(jnp.minimum(yi2, y2c) - jnp.maximum(yi1, y1c), 0.0))
            kill = (inter > tac + ai * c_thr) | (cm == first)
            cm = jnp.where(kill, _BIG, cm)
            keep = jnp.where(kio == s, first, keep)
            return s + 1, nk + 1, jnp.min(cm), cm, keep

        s, nk, _, _, keep = lax.while_loop(
            sel_cond, sel_body, (s, nk, jnp.min(cm0), cm0, keep))
        return fc + 1, s, nk, keep

    def chunk_cond(carry):
        fc, s, _, _ = carry
        return jnp.logical_and(fc < _R // 8, s < _TOP_K)

    _, _, _, keep = lax.while_loop(
        chunk_cond, chunk_body,
        (0, 0, 0, jnp.full((8, 128), _NP, jnp.int32)))
    ki_ref[...] = keep


def _sc_gather(table, idx):
    info = plsc.get_sparse_core_info()
    nw = info.num_cores * info.num_subcores
    bpw = _KB // nw
    mesh = plsc.VectorSubcoreMesh(core_axis_name="c", subcore_axis_name="s")

    @functools.partial(
        pl.kernel, mesh=mesh,
        out_type=jax.ShapeDtypeStruct((_KB, _KD), jnp.float32),
        scratch_types=[
            pltpu.VMEM((bpw,), jnp.int32),
            pltpu.VMEM((bpw, _KD), jnp.float32),
            pltpu.SemaphoreType.DMA,
        ],
    )
    def k(table_hbm, idx_hbm, out_hbm, idx_v, rows_v, sem):
        wid = lax.axis_index("s") * info.num_cores + lax.axis_index("c")
        base = wid * bpw
        pltpu.sync_copy(idx_hbm.at[pl.ds(base, bpw)], idx_v)
        pltpu.async_copy(table_hbm.at[idx_v], rows_v, sem).wait()
        pltpu.sync_copy(rows_v, out_hbm.at[pl.ds(base, bpw)])

    return k(table, idx)


def _comps(a):
    return a.T.reshape(a.shape[1], _R, _L)


def kernel(loc, conf, landms, prior_box):
    pad = _PP - _NP
    locp = jnp.pad(loc[0], ((0, pad), (0, 0)))
    confp = jnp.pad(conf[0], ((0, pad), (0, 0)))
    lmp = jnp.pad(landms[0], ((0, pad), (0, 0)))
    prp = jnp.pad(prior_box, ((0, pad), (0, 0)))
    confT = _comps(confp)
    inp = jnp.concatenate(
        [_comps(locp), _comps(prp), confT[1:2], _comps(lmp)], axis=0)

    comp, ki = pl.pallas_call(
        _decode_nms_body,
        out_shape=[
            jax.ShapeDtypeStruct((14, _R, _L), jnp.float32),
            jax.ShapeDtypeStruct((8, 128), jnp.int32),
        ],
        scratch_shapes=[pltpu.VMEM((_R, _L), jnp.int32)]
        + [pltpu.SMEM((_KB,), jnp.float32)] * 5,
    )(inp)

    table = jnp.concatenate([confT, comp], axis=0).reshape(
        16, _PP).T.reshape(_PP // 8, 128)
    idx = ki.reshape(-1)[:_KB]
    rows = _sc_gather(table, idx >> 3)
    sub = idx & 7
    r8 = rows.reshape(_KB, 8, 16)
    oh = (sub[:, None] == jnp.arange(8)[None, :]).astype(jnp.float32)
    sel = jnp.sum(r8 * oh[:, :, None], axis=1)
    conf_out = sel[:_TOP_K, 0:2]
    loc_out = sel[:_TOP_K, 2:6]
    lm_out = sel[:_TOP_K, 6:16]
    return conf_out, lm_out, loc_out


# speedup vs baseline: 1.2198x; 1.0321x over previous
import functools

import jax
import jax.numpy as jnp
from jax import lax
from jax.experimental import pallas as pl
from jax.experimental.pallas import tpu as pltpu
from jax.experimental.pallas import tpu_sc as plsc

_VAR0, _VAR1 = 0.1, 0.2
_CONF_THR = 0.5
_NMS_THR = 0.4
_TOP_K = 750
_NP = 20000
_PP = 20480
_R, _L = 160, 128
_SIZE = 640.0
_BIG = 2 ** 30
_KB = 768
_KD = 128


def _decode_nms_body(in_ref, comp_ref, ki_ref, miota_ref,
                     kx1_ref, ky1_ref, kx2_ref, ky2_ref, kta_ref):
    lx, ly, lw, lh = in_ref[0], in_ref[1], in_ref[2], in_ref[3]
    pcx, pcy, pw, ph = in_ref[4], in_ref[5], in_ref[6], in_ref[7]
    conf1 = in_ref[8]

    cx = pcx + lx * _VAR0 * pw
    cy = pcy + ly * _VAR0 * ph
    w = pw * jnp.exp(lw * _VAR1)
    h = ph * jnp.exp(lh * _VAR1)
    x1 = (cx - w / 2.0) * _SIZE
    y1 = (cy - h / 2.0) * _SIZE
    x2 = (cx + w / 2.0) * _SIZE
    y2 = (cy + h / 2.0) * _SIZE
    comp_ref[0] = x1
    comp_ref[1] = y1
    comp_ref[2] = x2
    comp_ref[3] = y2
    for k in range(5):
        comp_ref[4 + 2 * k] = (pcx + pw * in_ref[9 + 2 * k] * _VAR0) * _SIZE
        comp_ref[5 + 2 * k] = (pcy + ph * in_ref[9 + 2 * k + 1] * _VAR0) * _SIZE

    c_thr = _NMS_THR / (1.0 + _NMS_THR)
    c_eps = 1e-9 * c_thr

    iota2 = (lax.broadcasted_iota(jnp.int32, (_R, _L), 0) * _L
             + lax.broadcasted_iota(jnp.int32, (_R, _L), 1))
    lane_iota = lax.broadcasted_iota(jnp.int32, (1, _L), 1)
    kio = (lax.broadcasted_iota(jnp.int32, (8, 128), 0) * 128
           + lax.broadcasted_iota(jnp.int32, (8, 128), 1))

    miota_ref[...] = jnp.where(conf1 > _CONF_THR, iota2, _BIG)

    def ext_sel(first):
        row = first // _L
        lane = first - row * _L

        def ext(c):
            v = comp_ref[c, pl.ds(row, 1), :]
            return jnp.sum(jnp.where(lane_iota == lane, v, 0.0))

        return ext(0), ext(1), ext(2), ext(3)

    def chunk_body(carry):
        fc, s, nk, keep = carry
        rb = fc * 8
        x1c = comp_ref[0, pl.ds(rb, 8), :]
        y1c = comp_ref[1, pl.ds(rb, 8), :]
        x2c = comp_ref[2, pl.ds(rb, 8), :]
        y2c = comp_ref[3, pl.ds(rb, 8), :]
        tac = (jnp.maximum(x2c - x1c, 0.0) * jnp.maximum(y2c - y1c, 0.0)
               * c_thr + c_eps)

        def entry_test4(i, cm):
            for j in range(8):
                k = jnp.minimum(8 * i + j, nk - 1)
                xi1, yi1 = kx1_ref[k], ky1_ref[k]
                xi2, yi2 = kx2_ref[k], ky2_ref[k]
                inter = (jnp.maximum(jnp.minimum(xi2, x2c) - jnp.maximum(xi1, x1c), 0.0)
                         * jnp.maximum(jnp.minimum(yi2, y2c) - jnp.maximum(yi1, y1c), 0.0))
                cm = jnp.where(inter > tac + kta_ref[k], _BIG, cm)
            return cm

        cm0 = lax.fori_loop(0, (nk + 7) // 8, entry_test4,
                            miota_ref[pl.ds(rb, 8), :])

        def sel_cond(c):
            s, nk, first, cm, keep = c
            return jnp.logical_and(s < _TOP_K, first < _PP)

        def sel_body(c):
            s, nk, first, cm, keep = c
            xi1, yi1, xi2, yi2 = ext_sel(first)
            ai = jnp.maximum(xi2 - xi1, 0.0) * jnp.maximum(yi2 - yi1, 0.0)
            kx1_ref[nk] = xi1
            ky1_ref[nk] = yi1
            kx2_ref[nk] = xi2
            ky2_ref[nk] = yi2
            kta_ref[nk] = ai * c_thr
            inter = (jnp.maximum(jnp.minimum(xi2, x2c) - jnp.maximum(xi1, x1c), 0.0)
                     * jnp.maximum(jnp.minimum(yi2, y2c) - jnp.maximum(yi1, y1c), 0.0))
            kill = (inter > tac + ai * c_thr) | (cm == first)
            cm = jnp.where(kill, _BIG, cm)
            keep = jnp.where(kio == s, first, keep)
            return s + 1, nk + 1, jnp.min(cm), cm, keep

        s, nk, _, _, keep = lax.while_loop(
            sel_cond, sel_body, (s, nk, jnp.min(cm0), cm0, keep))
        return fc + 1, s, nk, keep

    def chunk_cond(carry):
        fc, s, _, _ = carry
        return jnp.logical_and(fc < _R // 8, s < _TOP_K)

    _, _, _, keep = lax.while_loop(
        chunk_cond, chunk_body,
        (0, 0, 0, jnp.full((8, 128), _NP, jnp.int32)))
    ki_ref[...] = keep


def _sc_gather(table, idx):
    info = plsc.get_sparse_core_info()
    nw = info.num_cores * info.num_subcores
    bpw = _KB // nw
    mesh = plsc.VectorSubcoreMesh(core_axis_name="c", subcore_axis_name="s")

    @functools.partial(
        pl.kernel, mesh=mesh,
        out_type=jax.ShapeDtypeStruct((_KB, _KD), jnp.float32),
        scratch_types=[
            pltpu.VMEM((bpw,), jnp.int32),
            pltpu.VMEM((bpw, _KD), jnp.float32),
            pltpu.SemaphoreType.DMA,
        ],
    )
    def k(table_hbm, idx_hbm, out_hbm, idx_v, rows_v, sem):
        wid = lax.axis_index("s") * info.num_cores + lax.axis_index("c")
        base = wid * bpw
        pltpu.sync_copy(idx_hbm.at[pl.ds(base, bpw)], idx_v)
        pltpu.async_copy(table_hbm.at[idx_v], rows_v, sem).wait()
        pltpu.sync_copy(rows_v, out_hbm.at[pl.ds(base, bpw)])

    return k(table, idx)


def _comps(a):
    return a.T.reshape(a.shape[1], _R, _L)


def kernel(loc, conf, landms, prior_box):
    pad = _PP - _NP
    locp = jnp.pad(loc[0], ((0, pad), (0, 0)))
    confp = jnp.pad(conf[0], ((0, pad), (0, 0)))
    lmp = jnp.pad(landms[0], ((0, pad), (0, 0)))
    prp = jnp.pad(prior_box, ((0, pad), (0, 0)))
    confT = _comps(confp)
    inp = jnp.concatenate(
        [_comps(locp), _comps(prp), confT[1:2], _comps(lmp)], axis=0)

    comp, ki = pl.pallas_call(
        _decode_nms_body,
        out_shape=[
            jax.ShapeDtypeStruct((14, _R, _L), jnp.float32),
            jax.ShapeDtypeStruct((8, 128), jnp.int32),
        ],
        scratch_shapes=[pltpu.VMEM((_R, _L), jnp.int32)]
        + [pltpu.SMEM((_KB,), jnp.float32)] * 5,
    )(inp)

    table = jnp.concatenate([confT, comp], axis=0).reshape(
        16, _PP).T.reshape(_PP // 8, 128)
    idx = ki.reshape(-1)[:_KB]
    rows = _sc_gather(table, idx >> 3)
    sub = idx & 7
    r8 = rows.reshape(_KB, 8, 16)
    oh = (sub[:, None] == jnp.arange(8)[None, :]).astype(jnp.float32)
    sel = jnp.sum(r8 * oh[:, :, None], axis=1)
    conf_out = sel[:_TOP_K, 0:2]
    loc_out = sel[:_TOP_K, 2:6]
    lm_out = sel[:_TOP_K, 6:16]
    return conf_out, lm_out, loc_out


# speedup vs baseline: 1.2373x; 1.0143x over previous
import functools

import jax
import jax.numpy as jnp
from jax import lax
from jax.experimental import pallas as pl
from jax.experimental.pallas import tpu as pltpu
from jax.experimental.pallas import tpu_sc as plsc

_VAR0, _VAR1 = 0.1, 0.2
_CONF_THR = 0.5
_NMS_THR = 0.4
_TOP_K = 750
_NP = 20000
_PP = 20480
_R, _L = 160, 128
_SIZE = 640.0
_BIG = 2 ** 30
_KB = 768
_KD = 128


def _decode_nms_body(in_ref, comp_ref, ki_ref, miota_ref,
                     kx1_ref, ky1_ref, kx2_ref, ky2_ref, kta_ref):
    lx, ly, lw, lh = in_ref[0], in_ref[1], in_ref[2], in_ref[3]
    pcx, pcy, pw, ph = in_ref[4], in_ref[5], in_ref[6], in_ref[7]
    conf1 = in_ref[8]

    cx = pcx + lx * _VAR0 * pw
    cy = pcy + ly * _VAR0 * ph
    w = pw * jnp.exp(lw * _VAR1)
    h = ph * jnp.exp(lh * _VAR1)
    x1 = (cx - w / 2.0) * _SIZE
    y1 = (cy - h / 2.0) * _SIZE
    x2 = (cx + w / 2.0) * _SIZE
    y2 = (cy + h / 2.0) * _SIZE
    comp_ref[0] = x1
    comp_ref[1] = y1
    comp_ref[2] = x2
    comp_ref[3] = y2
    for k in range(5):
        comp_ref[4 + 2 * k] = (pcx + pw * in_ref[9 + 2 * k] * _VAR0) * _SIZE
        comp_ref[5 + 2 * k] = (pcy + ph * in_ref[9 + 2 * k + 1] * _VAR0) * _SIZE

    c_thr = _NMS_THR / (1.0 + _NMS_THR)
    c_eps = 1e-9 * c_thr

    iota2 = (lax.broadcasted_iota(jnp.int32, (_R, _L), 0) * _L
             + lax.broadcasted_iota(jnp.int32, (_R, _L), 1))
    lane_iota = lax.broadcasted_iota(jnp.int32, (1, _L), 1)
    kio = (lax.broadcasted_iota(jnp.int32, (8, 128), 0) * 128
           + lax.broadcasted_iota(jnp.int32, (8, 128), 1))

    miota_ref[...] = jnp.where(conf1 > _CONF_THR, iota2, _BIG)

    def ext_sel(first):
        row = first // _L
        lane = first - row * _L

        def ext(c):
            v = comp_ref[c, pl.ds(row, 1), :]
            return jnp.sum(jnp.where(lane_iota == lane, v, 0.0))

        return ext(0), ext(1), ext(2), ext(3)

    def chunk_body(carry):
        fc, s, nk, keep = carry
        rb = fc * 8
        x1c = comp_ref[0, pl.ds(rb, 8), :]
        y1c = comp_ref[1, pl.ds(rb, 8), :]
        x2c = comp_ref[2, pl.ds(rb, 8), :]
        y2c = comp_ref[3, pl.ds(rb, 8), :]
        tac = (jnp.maximum(x2c - x1c, 0.0) * jnp.maximum(y2c - y1c, 0.0)
               * c_thr + c_eps)

        def entry_test4(i, cm):
            for j in range(16):
                k = jnp.minimum(16 * i + j, nk - 1)
                xi1, yi1 = kx1_ref[k], ky1_ref[k]
                xi2, yi2 = kx2_ref[k], ky2_ref[k]
                inter = (jnp.maximum(jnp.minimum(xi2, x2c) - jnp.maximum(xi1, x1c), 0.0)
                         * jnp.maximum(jnp.minimum(yi2, y2c) - jnp.maximum(yi1, y1c), 0.0))
                cm = jnp.where(inter > tac + kta_ref[k], _BIG, cm)
            return cm

        cm0 = lax.fori_loop(0, (nk + 15) // 16, entry_test4,
                            miota_ref[pl.ds(rb, 8), :])

        def sel_cond(c):
            s, nk, first, cm, keep = c
            return jnp.logical_and(s < _TOP_K, first < _PP)

        def sel_body(c):
            s, nk, first, cm, keep = c
            xi1, yi1, xi2, yi2 = ext_sel(first)
            ai = jnp.maximum(xi2 - xi1, 0.0) * jnp.maximum(yi2 - yi1, 0.0)
            kx1_ref[nk] = xi1
            ky1_ref[nk] = yi1
            kx2_ref[nk] = xi2
            ky2_ref[nk] = yi2
            kta_ref[nk] = ai * c_thr
            inter = (jnp.maximum(jnp.minimum(xi2, x2c) - jnp.maximum(xi1, x1c), 0.0)
                     * jnp.maximum(jnp.minimum(yi2, y2c) - jnp.maximum(yi1, y1c), 0.0))
            kill = (inter > tac + ai * c_thr) | (cm == first)
            cm = jnp.where(kill, _BIG, cm)
            keep = jnp.where(kio == s, first, keep)
            return s + 1, nk + 1, jnp.min(cm), cm, keep

        s, nk, _, _, keep = lax.while_loop(
            sel_cond, sel_body, (s, nk, jnp.min(cm0), cm0, keep))
        return fc + 1, s, nk, keep

    def chunk_cond(carry):
        fc, s, _, _ = carry
        return jnp.logical_and(fc < _R // 8, s < _TOP_K)

    _, _, _, keep = lax.while_loop(
        chunk_cond, chunk_body,
        (0, 0, 0, jnp.full((8, 128), _NP, jnp.int32)))
    ki_ref[...] = keep


def _sc_gather(table, idx):
    info = plsc.get_sparse_core_info()
    nw = info.num_cores * info.num_subcores
    bpw = _KB // nw
    mesh = plsc.VectorSubcoreMesh(core_axis_name="c", subcore_axis_name="s")

    @functools.partial(
        pl.kernel, mesh=mesh,
        out_type=jax.ShapeDtypeStruct((_KB, _KD), jnp.float32),
        scratch_types=[
            pltpu.VMEM((bpw,), jnp.int32),
            pltpu.VMEM((bpw, _KD), jnp.float32),
            pltpu.SemaphoreType.DMA,
        ],
    )
    def k(table_hbm, idx_hbm, out_hbm, idx_v, rows_v, sem):
        wid = lax.axis_index("s") * info.num_cores + lax.axis_index("c")
        base = wid * bpw
        pltpu.sync_copy(idx_hbm.at[pl.ds(base, bpw)], idx_v)
        pltpu.async_copy(table_hbm.at[idx_v], rows_v, sem).wait()
        pltpu.sync_copy(rows_v, out_hbm.at[pl.ds(base, bpw)])

    return k(table, idx)


def _comps(a):
    return a.T.reshape(a.shape[1], _R, _L)


def kernel(loc, conf, landms, prior_box):
    pad = _PP - _NP
    locp = jnp.pad(loc[0], ((0, pad), (0, 0)))
    confp = jnp.pad(conf[0], ((0, pad), (0, 0)))
    lmp = jnp.pad(landms[0], ((0, pad), (0, 0)))
    prp = jnp.pad(prior_box, ((0, pad), (0, 0)))
    confT = _comps(confp)
    inp = jnp.concatenate(
        [_comps(locp), _comps(prp), confT[1:2], _comps(lmp)], axis=0)

    comp, ki = pl.pallas_call(
        _decode_nms_body,
        out_shape=[
            jax.ShapeDtypeStruct((14, _R, _L), jnp.float32),
            jax.ShapeDtypeStruct((8, 128), jnp.int32),
        ],
        scratch_shapes=[pltpu.VMEM((_R, _L), jnp.int32)]
        + [pltpu.SMEM((_KB,), jnp.float32)] * 5,
    )(inp)

    table = jnp.concatenate([confT, comp], axis=0).reshape(
        16, _PP).T.reshape(_PP // 8, 128)
    idx = ki.reshape(-1)[:_KB]
    rows = _sc_gather(table, idx >> 3)
    sub = idx & 7
    r8 = rows.reshape(_KB, 8, 16)
    oh = (sub[:, None] == jnp.arange(8)[None, :]).astype(jnp.float32)
    sel = jnp.sum(r8 * oh[:, :, None], axis=1)
    conf_out = sel[:_TOP_K, 0:2]
    loc_out = sel[:_TOP_K, 2:6]
    lm_out = sel[:_TOP_K, 6:16]
    return conf_out, lm_out, loc_out
